# baseline (speedup 1.0000x reference)
import functools

import jax
import jax.numpy as jnp
from jax import lax
from jax.experimental import pallas as pl
from jax.experimental.pallas import tpu as pltpu
from jax.experimental.pallas import tpu_sc as plsc

_NC = 2
_NS = 16
_NW = _NC * _NS

_IDX_PER_STREAM = 128
_STREAMS_PER_CHUNK = 5
_CHUNK = _IDX_PER_STREAM * _STREAMS_PER_CHUNK


def _transpose_tc(w, vocab_pad, block_w):
    units = w.shape[0]

    def body(in_ref, out_ref):
        out_ref[...] = in_ref[...].T

    return pl.pallas_call(
        body,
        grid=(vocab_pad // block_w,),
        in_specs=[pl.BlockSpec((units, block_w), lambda i: (0, i))],
        out_specs=pl.BlockSpec((block_w, units), lambda i: (i, 0)),
        out_shape=jax.ShapeDtypeStruct((vocab_pad, units), w.dtype),
    )(w)


def _make_gather(vocab_pad, units, b):
    b_per_w = b // _NW
    n_chunks = b_per_w // _CHUNK
    n_pairs = n_chunks // 2
    rows = _STREAMS_PER_CHUNK

    mesh = plsc.VectorSubcoreMesh(core_axis_name="c", subcore_axis_name="s")

    @functools.partial(
        pl.kernel,
        mesh=mesh,
        compiler_params=pltpu.CompilerParams(use_tc_tiling_on_sc=False),
        out_type=jax.ShapeDtypeStruct((b, units), jnp.float32),
        scratch_types=[
            pltpu.VMEM((2, rows, _IDX_PER_STREAM), jnp.int32),
            pltpu.VMEM((2, _CHUNK, units), jnp.float32),
            pltpu.SemaphoreType.DMA((2,)),
        ],
    )
    def gather_kernel(table_hbm, idx_hbm, out_hbm, idx_v, rows_v, gat_sem):
        wid = lax.axis_index("s") * _NC + lax.axis_index("c")
        base_row = wid * (b_per_w // _IDX_PER_STREAM)
        out_base = wid * b_per_w

        def load_idx(g, h):
            pltpu.sync_copy(
                idx_hbm.at[pl.ds(base_row + g * rows, rows)], idx_v.at[h])

        def fire(h):
            for j in range(_STREAMS_PER_CHUNK):
                pltpu.async_copy(
                    table_hbm.at[idx_v.at[h, j]],
                    rows_v.at[h, pl.ds(j * _IDX_PER_STREAM, _IDX_PER_STREAM)],
                    gat_sem.at[h])

        def drain(h):
            for j in range(_STREAMS_PER_CHUNK):
                pltpu.make_async_copy(
                    table_hbm.at[idx_v.at[h, j]],
                    rows_v.at[h, pl.ds(j * _IDX_PER_STREAM, _IDX_PER_STREAM)],
                    gat_sem.at[h]).wait()

        def write(g, h):
            pltpu.sync_copy(rows_v.at[h],
                            out_hbm.at[pl.ds(out_base + g * _CHUNK, _CHUNK)])

        load_idx(0, 0)
        fire(0)

        def pair_body(k, _):
            g = 2 * k
            load_idx(g + 1, 1)
            drain(0)
            fire(1)
            write(g, 0)

            @pl.when(k + 1 < n_pairs)
            def _():
                load_idx(g + 2, 0)
            drain(1)

            @pl.when(k + 1 < n_pairs)
            def _():
                fire(0)
            write(g + 1, 1)
            return ()

        lax.fori_loop(0, n_pairs, pair_body, (), unroll=False)

    return gather_kernel


def kernel(inputs, kernel):
    units, vocab = kernel.shape
    batch, hist = inputs.shape
    b = batch * hist

    vocab_pad = 102400
    w = jnp.pad(kernel, ((0, 0), (0, vocab_pad - vocab)))
    table = _transpose_tc(w, vocab_pad, block_w=4096)

    n_split = 4
    bs = batch // n_split
    gather = _make_gather(vocab_pad, units, bs * hist)
    pieces = []
    for i in range(n_split):
        idx = inputs[i * bs:(i + 1) * bs].astype(jnp.int32)
        idx = idx.reshape(bs * hist // _IDX_PER_STREAM, _IDX_PER_STREAM)
        pieces.append(gather(table, idx).reshape(bs, hist, units))
    return jnp.concatenate(pieces, axis=0)


# speedup vs baseline: 1.0445x; 1.0445x over previous
import functools

import jax
import jax.numpy as jnp
from jax import lax
from jax.experimental import pallas as pl
from jax.experimental.pallas import tpu as pltpu
from jax.experimental.pallas import tpu_sc as plsc

_NC = 2
_NS = 16
_NW = _NC * _NS

_IDX_PER_STREAM = 128
_STREAMS_PER_CHUNK = 4
_CHUNK = _IDX_PER_STREAM * _STREAMS_PER_CHUNK


def _transpose_tc(w, vocab_pad, block_w):
    units = w.shape[0]

    def body(in_ref, out_ref):
        out_ref[...] = in_ref[...].T

    return pl.pallas_call(
        body,
        grid=(vocab_pad // block_w,),
        in_specs=[pl.BlockSpec((units, block_w), lambda i: (0, i))],
        out_specs=pl.BlockSpec((block_w, units), lambda i: (i, 0)),
        out_shape=jax.ShapeDtypeStruct((vocab_pad, units), w.dtype),
    )(w)


def _make_gather(vocab_pad, units, b):
    b_per_w = b // _NW
    n_chunks = b_per_w // _CHUNK
    n_pairs = n_chunks // 2
    rows = _STREAMS_PER_CHUNK

    mesh = plsc.VectorSubcoreMesh(core_axis_name="c", subcore_axis_name="s")

    @functools.partial(
        pl.kernel,
        mesh=mesh,
        compiler_params=pltpu.CompilerParams(use_tc_tiling_on_sc=False),
        out_type=jax.ShapeDtypeStruct((b, units), jnp.float32),
        scratch_types=[
            pltpu.VMEM((2, rows, _IDX_PER_STREAM), jnp.int32),
            pltpu.VMEM((2, _CHUNK, units), jnp.float32),
            pltpu.SemaphoreType.DMA((2,)),
        ],
    )
    def gather_kernel(table_hbm, idx_hbm, out_hbm, idx_v, rows_v, gat_sem):
        wid = lax.axis_index("s") * _NC + lax.axis_index("c")
        base_row = wid * (b_per_w // _IDX_PER_STREAM)
        out_base = wid * b_per_w

        def load_idx(g, h):
            pltpu.sync_copy(
                idx_hbm.at[pl.ds(base_row + g * rows, rows)], idx_v.at[h])

        def fire(h):
            for j in range(_STREAMS_PER_CHUNK):
                pltpu.async_copy(
                    table_hbm.at[idx_v.at[h, j]],
                    rows_v.at[h, pl.ds(j * _IDX_PER_STREAM, _IDX_PER_STREAM)],
                    gat_sem.at[h])

        def drain(h):
            for j in range(_STREAMS_PER_CHUNK):
                pltpu.make_async_copy(
                    table_hbm.at[idx_v.at[h, j]],
                    rows_v.at[h, pl.ds(j * _IDX_PER_STREAM, _IDX_PER_STREAM)],
                    gat_sem.at[h]).wait()

        def write(g, h):
            pltpu.sync_copy(rows_v.at[h],
                            out_hbm.at[pl.ds(out_base + g * _CHUNK, _CHUNK)])

        load_idx(0, 0)
        fire(0)

        def pair_body(k, _):
            g = 2 * k
            load_idx(g + 1, 1)
            drain(0)
            fire(1)
            write(g, 0)

            @pl.when(k + 1 < n_pairs)
            def _():
                load_idx(g + 2, 0)
            drain(1)

            @pl.when(k + 1 < n_pairs)
            def _():
                fire(0)
            write(g + 1, 1)
            return ()

        lax.fori_loop(0, n_pairs, pair_body, (), unroll=False)

    return gather_kernel


def kernel(inputs, kernel):
    units, vocab = kernel.shape
    batch, hist = inputs.shape
    b = batch * hist

    vocab_pad = 102400
    w = jnp.pad(kernel, ((0, 0), (0, vocab_pad - vocab)))
    table = _transpose_tc(w, vocab_pad, block_w=4096)

    idx = inputs.astype(jnp.int32).reshape(b // _IDX_PER_STREAM,
                                           _IDX_PER_STREAM)
    out = _make_gather(vocab_pad, units, b)(table, idx)
    return out.reshape(batch, hist, units)


# speedup vs baseline: 1.0480x; 1.0033x over previous
import functools

import jax
import jax.numpy as jnp
from jax import lax
from jax.experimental import pallas as pl
from jax.experimental.pallas import tpu as pltpu
from jax.experimental.pallas import tpu_sc as plsc

_NC = 2
_NS = 16
_NW = _NC * _NS

_IDX_PER_STREAM = 128
_STREAMS_PER_CHUNK = 5
_CHUNK = _IDX_PER_STREAM * _STREAMS_PER_CHUNK


def _transpose_tc(w, vocab_pad, block_w):
    units = w.shape[0]

    def body(in_ref, out_ref):
        out_ref[...] = in_ref[...].T

    return pl.pallas_call(
        body,
        grid=(vocab_pad // block_w,),
        in_specs=[pl.BlockSpec((units, block_w), lambda i: (0, i))],
        out_specs=pl.BlockSpec((block_w, units), lambda i: (i, 0)),
        out_shape=jax.ShapeDtypeStruct((vocab_pad, units), w.dtype),
    )(w)


def _make_gather(vocab_pad, units, b):
    b_per_w = b // _NW
    n_chunks = b_per_w // _CHUNK
    n_pairs = n_chunks // 2
    rows = _STREAMS_PER_CHUNK

    mesh = plsc.VectorSubcoreMesh(core_axis_name="c", subcore_axis_name="s")

    @functools.partial(
        pl.kernel,
        mesh=mesh,
        compiler_params=pltpu.CompilerParams(use_tc_tiling_on_sc=False),
        out_type=jax.ShapeDtypeStruct((b, units), jnp.float32),
        scratch_types=[
            pltpu.VMEM((b_per_w // _IDX_PER_STREAM, _IDX_PER_STREAM),
                       jnp.int32),
            pltpu.VMEM((2, _CHUNK, units), jnp.float32),
            pltpu.SemaphoreType.DMA((2,)),
        ],
    )
    def gather_kernel(table_hbm, idx_hbm, out_hbm, idx_v, rows_v, gat_sem):
        wid = lax.axis_index("s") * _NC + lax.axis_index("c")
        base_row = wid * (b_per_w // _IDX_PER_STREAM)
        out_base = wid * b_per_w

        def fire(g, h):
            for j in range(_STREAMS_PER_CHUNK):
                pltpu.async_copy(
                    table_hbm.at[idx_v.at[g * rows + j]],
                    rows_v.at[h, pl.ds(j * _IDX_PER_STREAM, _IDX_PER_STREAM)],
                    gat_sem.at[h])

        def drain(g, h):
            for j in range(_STREAMS_PER_CHUNK):
                pltpu.make_async_copy(
                    table_hbm.at[idx_v.at[g * rows + j]],
                    rows_v.at[h, pl.ds(j * _IDX_PER_STREAM, _IDX_PER_STREAM)],
                    gat_sem.at[h]).wait()

        def write(g, h):
            pltpu.sync_copy(rows_v.at[h],
                            out_hbm.at[pl.ds(out_base + g * _CHUNK, _CHUNK)])

        pltpu.sync_copy(
            idx_hbm.at[pl.ds(base_row, b_per_w // _IDX_PER_STREAM)], idx_v)
        fire(0, 0)

        def pair_body(k, _):
            g = 2 * k
            drain(g, 0)
            fire(g + 1, 1)
            write(g, 0)
            drain(g + 1, 1)

            @pl.when(k + 1 < n_pairs)
            def _():
                fire(g + 2, 0)
            write(g + 1, 1)
            return ()

        lax.fori_loop(0, n_pairs, pair_body, (), unroll=False)

    return gather_kernel


def kernel(inputs, kernel):
    units, vocab = kernel.shape
    batch, hist = inputs.shape
    b = batch * hist

    vocab_pad = 102400
    w = jnp.pad(kernel, ((0, 0), (0, vocab_pad - vocab)))
    table = _transpose_tc(w, vocab_pad, block_w=4096)

    idx = inputs.astype(jnp.int32).reshape(b // _IDX_PER_STREAM,
                                           _IDX_PER_STREAM)
    out = _make_gather(vocab_pad, units, b)(table, idx)
    return out.reshape(batch, hist, units)


# speedup vs baseline: 1.0720x; 1.0229x over previous
import functools

import jax
import jax.numpy as jnp
from jax import lax
from jax.experimental import pallas as pl
from jax.experimental.pallas import tpu as pltpu
from jax.experimental.pallas import tpu_sc as plsc

_NC = 2
_NS = 16
_NW = _NC * _NS

_IDX_PER_STREAM = 128
_STREAMS_PER_CHUNK = 5
_CHUNK = _IDX_PER_STREAM * _STREAMS_PER_CHUNK


def _transpose_tc(w, vocab_pad, block_w):
    units = w.shape[0]

    def body(in_ref, out_ref):
        out_ref[...] = in_ref[...].T

    return pl.pallas_call(
        body,
        grid=(vocab_pad // block_w,),
        in_specs=[pl.BlockSpec((units, block_w), lambda i: (0, i))],
        out_specs=pl.BlockSpec((block_w, units), lambda i: (i, 0)),
        out_shape=jax.ShapeDtypeStruct((vocab_pad, units), w.dtype),
    )(w)


def _make_gather(vocab_pad, units, b):
    b_per_w = b // _NW
    n_chunks = b_per_w // _CHUNK
    n_pairs = n_chunks // 2
    rows = _STREAMS_PER_CHUNK

    mesh = plsc.VectorSubcoreMesh(core_axis_name="c", subcore_axis_name="s")

    @functools.partial(
        pl.kernel,
        mesh=mesh,
        compiler_params=pltpu.CompilerParams(use_tc_tiling_on_sc=False),
        out_type=jax.ShapeDtypeStruct((b, units), jnp.float32),
        scratch_types=[
            pltpu.VMEM((b_per_w // _IDX_PER_STREAM, _IDX_PER_STREAM),
                       jnp.int32),
            pltpu.VMEM((2, _CHUNK, units), jnp.float32),
            pltpu.SemaphoreType.DMA((2,)),
        ],
    )
    def gather_kernel(table_hbm, idx_hbm, out_hbm, idx_v, rows_v, gat_sem):
        wid = lax.axis_index("s") * _NC + lax.axis_index("c")
        base_row = wid * (b_per_w // _IDX_PER_STREAM)
        out_base = wid * b_per_w

        def fire(g, h):
            for j in range(_STREAMS_PER_CHUNK):
                pltpu.async_copy(
                    table_hbm.at[idx_v.at[g * rows + j]],
                    rows_v.at[h, pl.ds(j * _IDX_PER_STREAM, _IDX_PER_STREAM)],
                    gat_sem.at[h])

        def drain(g, h):
            for j in range(_STREAMS_PER_CHUNK):
                pltpu.make_async_copy(
                    table_hbm.at[idx_v.at[g * rows + j]],
                    rows_v.at[h, pl.ds(j * _IDX_PER_STREAM, _IDX_PER_STREAM)],
                    gat_sem.at[h]).wait()

        def write(g, h):
            pltpu.sync_copy(rows_v.at[h],
                            out_hbm.at[pl.ds(out_base + g * _CHUNK, _CHUNK)])

        pltpu.sync_copy(
            idx_hbm.at[pl.ds(base_row, b_per_w // _IDX_PER_STREAM)], idx_v)
        fire(0, 0)

        def pair_body(k, _):
            g = 2 * k
            drain(g, 0)
            fire(g + 1, 1)
            write(g, 0)
            drain(g + 1, 1)

            @pl.when(k + 1 < n_pairs)
            def _():
                fire(g + 2, 0)
            write(g + 1, 1)
            return ()

        lax.fori_loop(0, n_pairs, pair_body, (), unroll=False)

    return gather_kernel


def kernel(inputs, kernel):
    units, vocab = kernel.shape
    batch, hist = inputs.shape
    b = batch * hist

    vocab_pad = 102400
    table = _transpose_tc(kernel, vocab_pad, block_w=4096)

    idx = inputs.astype(jnp.int32).reshape(b // _IDX_PER_STREAM,
                                           _IDX_PER_STREAM)
    out = _make_gather(vocab_pad, units, b)(table, idx)
    return out.reshape(batch, hist, units)
